# Initial kernel scaffold; baseline (speedup 1.0000x reference)
import jax
import jax.numpy as jnp
from jax.experimental import pallas as pl


def kernel(x, positions, topk, W, a_src, a_dst, ln_gamma, ln_beta):
    raise NotImplementedError("write your pallas kernel here")


# speedup vs baseline: 17.8794x; 17.8794x over previous
import functools

import jax
import jax.numpy as jnp
from jax import lax
from jax.experimental import pallas as pl
from jax.experimental.pallas import tpu as pltpu
from jax.experimental.pallas import tpu_sc as plsc

K = 16
NW = 32
GN = 8
ROWS = GN * K


def _proj_body(x_ref, w_ref, out_ref):
    z = jnp.dot(x_ref[...], w_ref[...], preferred_element_type=jnp.float32)
    lane = lax.broadcasted_iota(jnp.int32, z.shape, 1)
    out_ref[...] = jnp.where((lane >= 128) & (z < 0.0), 0.2 * z, z)


def _knn_body(col_ref, row_ref, out_ref, *, npad, rb):
    def _rq(v):
        return v.astype(jnp.bfloat16).astype(jnp.float32)

    q0 = _rq(col_ref[0:1, :])
    q1 = _rq(col_ref[1:2, :])
    q2 = _rq(col_ref[2:3, :])
    sqc = col_ref[3:4, :]
    p0 = _rq(row_ref[:, 0:1])
    p1 = _rq(row_ref[:, 1:2])
    p2 = _rq(row_ref[:, 2:3])
    sqr = row_ref[:, 3:4]
    dot = p0 * q0 + p1 * q1 + p2 * q2
    scores = (sqr + sqc) - 2.0 * dot
    coli = lax.broadcasted_iota(jnp.int32, (rb, npad), 1)
    big = jnp.int32(2**30)
    for kk in range(K):
        m = jnp.min(scores, axis=1, keepdims=True)
        cand = jnp.where(scores <= m, coli, big)
        a = jnp.min(cand, axis=1, keepdims=True)
        out_ref[:, kk:kk + 1] = a
        scores = jnp.where(coli == a, jnp.float32(jnp.inf), scores)


def _ln_body(hp_ref, x_ref, g_ref, b_ref, out_ref):
    y = hp_ref[...] + x_ref[...]
    mu = jnp.mean(y, axis=1, keepdims=True)
    yc = y - mu
    var = jnp.mean(yc * yc, axis=1, keepdims=True)
    out_ref[...] = yc / jnp.sqrt(var + 1e-5) * g_ref[0:1, :] + b_ref[0:1, :]


def _make_agg(npad):
    per_w = npad // NW
    ng = per_w // GN
    mesh = plsc.VectorSubcoreMesh(
        core_axis_name="c", subcore_axis_name="s", num_cores=2,
        num_subcores=16)

    @functools.partial(
        pl.kernel,
        mesh=mesh,
        compiler_params=pltpu.CompilerParams(needs_layout_passes=False),
        out_type=jax.ShapeDtypeStruct((npad, 128), jnp.float32),
        scratch_types=[
            pltpu.VMEM((ng, ROWS), jnp.int32),
            pltpu.VMEM((ROWS, 256), jnp.float32),
            pltpu.VMEM((GN, 128), jnp.float32),
            pltpu.SemaphoreType.DMA,
        ],
    )
    def agg(hs_hbm, idx_hbm, out_hbm, idx_v, rows_v, out_v, sem):
        wid = lax.axis_index("s") * 2 + lax.axis_index("c")
        base = wid * per_w
        pltpu.sync_copy(idx_hbm.at[wid], idx_v)

        def node_body(j, carry):
            rb = j * K
            rid = lax.iota(jnp.int32, K) + rb
            for h in range(4):
                col = jnp.full((K,), 128 + 32 * h, jnp.int32)
                sg = plsc.load_gather(rows_v, [rid, col])
                m = jnp.max(sg)
                p = jnp.exp(sg - m)
                pn = p / jnp.sum(p)
                for t in range(2):
                    c0 = 32 * h + 16 * t
                    acc = jnp.zeros((K,), jnp.float32)
                    for kk in range(K):
                        acc = acc + pn[kk] * rows_v[rb + kk, pl.ds(c0, 16)]
                    out_v[j, pl.ds(c0, 16)] = acc
            return carry

        def body(g, carry):
            pltpu.async_copy(hs_hbm.at[idx_v.at[g]], rows_v, sem).wait()
            lax.fori_loop(0, GN, node_body, 0)
            pltpu.sync_copy(out_v, out_hbm.at[pl.ds(base + g * GN, GN)])
            return carry

        lax.fori_loop(0, ng, body, 0)

    return agg


def kernel(x, positions, topk, W, a_src, a_dst, ln_gamma, ln_beta):
    B, N, F = x.shape
    H, D = a_src.shape
    npad = ((N + 255) // 256) * 256

    x2 = jnp.pad(x[0], ((0, npad - N), (0, 0)))
    pos = jnp.pad(positions[0], ((0, npad - N), (0, 0)))
    sq = jnp.sum(pos * pos, axis=1)
    sqcol = jnp.where(jnp.arange(npad) < N, sq, 1e30)
    colinfo = jnp.concatenate(
        [pos.T, sqcol[None, :], jnp.zeros((4, npad), jnp.float32)], axis=0)
    rowinfo = jnp.concatenate(
        [pos, sq[:, None], jnp.zeros((npad, 4), jnp.float32)], axis=1)

    avec = (a_src + a_dst).reshape(H * D)
    sel = (jnp.arange(H * D)[:, None] // D == jnp.arange(H)[None, :])
    W2 = (W * avec[None, :]) @ sel.astype(jnp.float32)
    Wsc = W2[:, jnp.arange(H * D) // D]
    Wbig = jnp.concatenate([W, Wsc], axis=1)

    rb2 = 512
    hs = pl.pallas_call(
        _proj_body,
        grid=(npad // rb2,),
        in_specs=[
            pl.BlockSpec((rb2, F), lambda i: (i, 0)),
            pl.BlockSpec((F, 256), lambda i: (0, 0)),
        ],
        out_specs=pl.BlockSpec((rb2, 256), lambda i: (i, 0)),
        out_shape=jax.ShapeDtypeStruct((npad, 256), jnp.float32),
    )(x2, Wbig)

    rb = 256
    idx = pl.pallas_call(
        functools.partial(_knn_body, npad=npad, rb=rb),
        grid=(npad // rb,),
        in_specs=[
            pl.BlockSpec((8, npad), lambda i: (0, 0)),
            pl.BlockSpec((rb, 8), lambda i: (i, 0)),
        ],
        out_specs=pl.BlockSpec((rb, K), lambda i: (i, 0)),
        out_shape=jax.ShapeDtypeStruct((npad, K), jnp.int32),
    )(colinfo, rowinfo)

    idxr = idx.reshape(NW, npad // NW // GN, ROWS)
    hp = _make_agg(npad)(hs, idxr)

    g2 = jnp.tile(ln_gamma[None, :], (8, 1))
    b2 = jnp.tile(ln_beta[None, :], (8, 1))
    out = pl.pallas_call(
        _ln_body,
        grid=(npad // rb2,),
        in_specs=[
            pl.BlockSpec((rb2, F), lambda i: (i, 0)),
            pl.BlockSpec((rb2, F), lambda i: (i, 0)),
            pl.BlockSpec((8, F), lambda i: (0, 0)),
            pl.BlockSpec((8, F), lambda i: (0, 0)),
        ],
        out_specs=pl.BlockSpec((rb2, F), lambda i: (i, 0)),
        out_shape=jax.ShapeDtypeStruct((npad, F), jnp.float32),
    )(hp, x2, g2, b2)

    return out[:N].reshape(B, N, H * D)


# speedup vs baseline: 18.1217x; 1.0136x over previous
import functools

import jax
import jax.numpy as jnp
from jax import lax
from jax.experimental import pallas as pl
from jax.experimental.pallas import tpu as pltpu
from jax.experimental.pallas import tpu_sc as plsc

K = 16
NW = 32
GN = 8
ROWS = GN * K


def _proj_body(x_ref, w_ref, out_ref):
    z = jnp.dot(x_ref[...], w_ref[...], preferred_element_type=jnp.float32)
    lane = lax.broadcasted_iota(jnp.int32, z.shape, 1)
    out_ref[...] = jnp.where((lane >= 128) & (z < 0.0), 0.2 * z, z)


def _knn_body(col_ref, row_ref, out_ref, *, npad, rb):
    def _rq(v):
        return v.astype(jnp.bfloat16).astype(jnp.float32)

    q0 = _rq(col_ref[0:1, :])
    q1 = _rq(col_ref[1:2, :])
    q2 = _rq(col_ref[2:3, :])
    sqc = col_ref[3:4, :]
    p0 = _rq(row_ref[:, 0:1])
    p1 = _rq(row_ref[:, 1:2])
    p2 = _rq(row_ref[:, 2:3])
    sqr = row_ref[:, 3:4]
    dot = p0 * q0 + p1 * q1 + p2 * q2
    scores = (sqr + sqc) - 2.0 * dot
    coli = lax.broadcasted_iota(jnp.int32, (rb, npad), 1)
    lane = lax.broadcasted_iota(jnp.int32, (rb, 128), 1)
    nch = npad // 128
    big = jnp.int32(2**30)
    for kk in range(K):
        v = scores[:, 0:128]
        ii = lane
        for c in range(1, nch):
            ch = scores[:, c * 128:(c + 1) * 128]
            lt = ch < v
            v = jnp.where(lt, ch, v)
            ii = jnp.where(lt, lane + c * 128, ii)
        mv = jnp.min(v, axis=1, keepdims=True)
        a = jnp.min(jnp.where(v <= mv, ii, big), axis=1, keepdims=True)
        out_ref[:, kk:kk + 1] = a
        scores = jnp.where(coli == a, jnp.float32(jnp.inf), scores)


def _ln_body(hp_ref, x_ref, g_ref, b_ref, out_ref):
    y = hp_ref[...] + x_ref[...]
    mu = jnp.mean(y, axis=1, keepdims=True)
    yc = y - mu
    var = jnp.mean(yc * yc, axis=1, keepdims=True)
    out_ref[...] = yc / jnp.sqrt(var + 1e-5) * g_ref[0:1, :] + b_ref[0:1, :]


def _make_agg(npad):
    per_w = npad // NW
    ng = per_w // GN
    mesh = plsc.VectorSubcoreMesh(
        core_axis_name="c", subcore_axis_name="s", num_cores=2,
        num_subcores=16)

    @functools.partial(
        pl.kernel,
        mesh=mesh,
        compiler_params=pltpu.CompilerParams(needs_layout_passes=False),
        out_type=jax.ShapeDtypeStruct((npad, 128), jnp.float32),
        scratch_types=[
            pltpu.VMEM((ng, ROWS), jnp.int32),
            pltpu.VMEM((ROWS, 256), jnp.float32),
            pltpu.VMEM((ROWS, 256), jnp.float32),
            pltpu.VMEM((GN, 128), jnp.float32),
            pltpu.SemaphoreType.DMA,
            pltpu.SemaphoreType.DMA,
        ],
    )
    def agg(hs_hbm, idx_hbm, out_hbm, idx_v, rows0, rows1, out_v, sem0,
            sem1):
        wid = lax.axis_index("s") * 2 + lax.axis_index("c")
        base = wid * per_w
        pltpu.sync_copy(idx_hbm.at[wid], idx_v)
        pltpu.async_copy(hs_hbm.at[idx_v.at[0]], rows0, sem0)

        def make_node_body(rows_v):
            def node_body(j, carry):
                rb = j * K
                rid = lax.iota(jnp.int32, K) + rb
                for h in range(4):
                    col = jnp.full((K,), 128 + 32 * h, jnp.int32)
                    sg = plsc.load_gather(rows_v, [rid, col])
                    m = jnp.max(sg)
                    p = jnp.exp(sg - m)
                    pn = p / jnp.sum(p)
                    for t in range(2):
                        c0 = 32 * h + 16 * t
                        acc = jnp.zeros((K,), jnp.float32)
                        for kk in range(K):
                            acc = acc + pn[kk] * rows_v[rb + kk,
                                                        pl.ds(c0, 16)]
                        out_v[j, pl.ds(c0, 16)] = acc
                return carry
            return node_body

        def pair_body(g2, carry):
            for rows, sem, nrows, nsem, b in ((rows0, sem0, rows1, sem1, 0),
                                              (rows1, sem1, rows0, sem0, 1)):
                g = g2 * 2 + b

                @pl.when(g + 1 < ng)
                def _():
                    pltpu.async_copy(hs_hbm.at[idx_v.at[g + 1]], nrows,
                                     nsem)

                pltpu.make_async_copy(hs_hbm.at[idx_v.at[g]], rows,
                                      sem).wait()
                lax.fori_loop(0, GN, make_node_body(rows), 0)
                pltpu.sync_copy(out_v, out_hbm.at[pl.ds(base + g * GN, GN)])
            return carry

        lax.fori_loop(0, ng // 2, pair_body, 0)

    return agg


def kernel(x, positions, topk, W, a_src, a_dst, ln_gamma, ln_beta):
    B, N, F = x.shape
    H, D = a_src.shape
    npad = ((N + 255) // 256) * 256

    x2 = jnp.pad(x[0], ((0, npad - N), (0, 0)))
    pos = jnp.pad(positions[0], ((0, npad - N), (0, 0)))
    sq = jnp.sum(pos * pos, axis=1)
    sqcol = jnp.where(jnp.arange(npad) < N, sq, 1e30)
    colinfo = jnp.concatenate(
        [pos.T, sqcol[None, :], jnp.zeros((4, npad), jnp.float32)], axis=0)
    rowinfo = jnp.concatenate(
        [pos, sq[:, None], jnp.zeros((npad, 4), jnp.float32)], axis=1)

    avec = (a_src + a_dst).reshape(H * D)
    sel = (jnp.arange(H * D)[:, None] // D == jnp.arange(H)[None, :])
    W2 = (W * avec[None, :]) @ sel.astype(jnp.float32)
    Wsc = W2[:, jnp.arange(H * D) // D]
    Wbig = jnp.concatenate([W, Wsc], axis=1)

    rb2 = 512
    hs = pl.pallas_call(
        _proj_body,
        grid=(npad // rb2,),
        in_specs=[
            pl.BlockSpec((rb2, F), lambda i: (i, 0)),
            pl.BlockSpec((F, 256), lambda i: (0, 0)),
        ],
        out_specs=pl.BlockSpec((rb2, 256), lambda i: (i, 0)),
        out_shape=jax.ShapeDtypeStruct((npad, 256), jnp.float32),
    )(x2, Wbig)

    rb = 256
    idx = pl.pallas_call(
        functools.partial(_knn_body, npad=npad, rb=rb),
        grid=(npad // rb,),
        in_specs=[
            pl.BlockSpec((8, npad), lambda i: (0, 0)),
            pl.BlockSpec((rb, 8), lambda i: (i, 0)),
        ],
        out_specs=pl.BlockSpec((rb, K), lambda i: (i, 0)),
        out_shape=jax.ShapeDtypeStruct((npad, K), jnp.int32),
    )(colinfo, rowinfo)

    idxr = idx.reshape(NW, npad // NW // GN, ROWS)
    hp = _make_agg(npad)(hs, idxr)

    g2 = jnp.tile(ln_gamma[None, :], (8, 1))
    b2 = jnp.tile(ln_beta[None, :], (8, 1))
    out = pl.pallas_call(
        _ln_body,
        grid=(npad // rb2,),
        in_specs=[
            pl.BlockSpec((rb2, F), lambda i: (i, 0)),
            pl.BlockSpec((rb2, F), lambda i: (i, 0)),
            pl.BlockSpec((8, F), lambda i: (0, 0)),
            pl.BlockSpec((8, F), lambda i: (0, 0)),
        ],
        out_specs=pl.BlockSpec((rb2, F), lambda i: (i, 0)),
        out_shape=jax.ShapeDtypeStruct((npad, F), jnp.float32),
    )(hp, x2, g2, b2)

    return out[:N].reshape(B, N, H * D)


# speedup vs baseline: 19.9678x; 1.1019x over previous
import functools

import jax
import jax.numpy as jnp
from jax import lax
from jax.experimental import pallas as pl
from jax.experimental.pallas import tpu as pltpu
from jax.experimental.pallas import tpu_sc as plsc

K = 16
NW = 32
GN = 8
ROWS = GN * K


def _proj_body(x_ref, w_ref, out_ref):
    z = jnp.dot(x_ref[...], w_ref[...], preferred_element_type=jnp.float32)
    lane = lax.broadcasted_iota(jnp.int32, z.shape, 1)
    out_ref[...] = jnp.where((lane >= 128) & (z < 0.0), 0.2 * z, z)


def _knn_body(col_ref, row_ref, out_ref, *, npad, rb):
    def _rq(v):
        return v.astype(jnp.bfloat16).astype(jnp.float32)

    q0 = _rq(col_ref[0:1, :])
    q1 = _rq(col_ref[1:2, :])
    q2 = _rq(col_ref[2:3, :])
    sqc = col_ref[3:4, :]
    p0 = _rq(row_ref[:, 0:1])
    p1 = _rq(row_ref[:, 1:2])
    p2 = _rq(row_ref[:, 2:3])
    sqr = row_ref[:, 3:4]
    dot = p0 * q0 + p1 * q1 + p2 * q2
    scores = (sqr + sqc) - 2.0 * dot
    coli = lax.broadcasted_iota(jnp.int32, (rb, npad), 1)
    nch = npad // 128
    big = jnp.int32(2**30)
    for kk in range(K):
        v = scores[:, 0:128]
        ii = coli[:, 0:128]
        for c in range(1, nch):
            ch = scores[:, c * 128:(c + 1) * 128]
            lt = ch < v
            v = jnp.where(lt, ch, v)
            ii = jnp.where(lt, coli[:, c * 128:(c + 1) * 128], ii)
        mv = jnp.min(v, axis=1, keepdims=True)
        a = jnp.min(jnp.where(v <= mv, ii, big), axis=1, keepdims=True)
        out_ref[:, kk:kk + 1] = a
        scores = jnp.where(coli == a, jnp.float32(jnp.inf), scores)


def _ln_body(hp_ref, x_ref, g_ref, b_ref, out_ref):
    y = hp_ref[...] + x_ref[...]
    mu = jnp.mean(y, axis=1, keepdims=True)
    yc = y - mu
    var = jnp.mean(yc * yc, axis=1, keepdims=True)
    out_ref[...] = yc / jnp.sqrt(var + 1e-5) * g_ref[0:1, :] + b_ref[0:1, :]


def _make_agg(npad):
    per_w = npad // NW
    ng = per_w // GN
    mesh = plsc.VectorSubcoreMesh(
        core_axis_name="c", subcore_axis_name="s", num_cores=2,
        num_subcores=16)

    @functools.partial(
        pl.kernel,
        mesh=mesh,
        compiler_params=pltpu.CompilerParams(needs_layout_passes=False),
        out_type=jax.ShapeDtypeStruct((npad, 128), jnp.float32),
        scratch_types=[
            pltpu.VMEM((ng, ROWS), jnp.int32),
            pltpu.VMEM((ROWS, 256), jnp.float32),
            pltpu.VMEM((ROWS, 256), jnp.float32),
            pltpu.VMEM((GN, 128), jnp.float32),
            pltpu.SemaphoreType.DMA,
            pltpu.SemaphoreType.DMA,
        ],
    )
    def agg(hs_hbm, idx_hbm, out_hbm, idx_v, rows0, rows1, out_v, sem0,
            sem1):
        wid = lax.axis_index("s") * 2 + lax.axis_index("c")
        base = wid * per_w
        pltpu.sync_copy(idx_hbm.at[wid], idx_v)
        pltpu.async_copy(hs_hbm.at[idx_v.at[0]], rows0, sem0)

        def make_node_body(rows_v):
            def node_body(j, carry):
                rb = j * K
                rid = lax.iota(jnp.int32, K) + rb
                for h in range(4):
                    col = jnp.full((K,), 128 + 32 * h, jnp.int32)
                    sg = plsc.load_gather(rows_v, [rid, col])
                    m = jnp.max(sg)
                    p = jnp.exp(sg - m)
                    pn = p / jnp.sum(p)
                    for t in range(2):
                        c0 = 32 * h + 16 * t
                        acc = jnp.zeros((K,), jnp.float32)
                        for kk in range(K):
                            acc = acc + pn[kk] * rows_v[rb + kk,
                                                        pl.ds(c0, 16)]
                        out_v[j, pl.ds(c0, 16)] = acc
                return carry
            return node_body

        def pair_body(g2, carry):
            for rows, sem, nrows, nsem, b in ((rows0, sem0, rows1, sem1, 0),
                                              (rows1, sem1, rows0, sem0, 1)):
                g = g2 * 2 + b

                @pl.when(g + 1 < ng)
                def _():
                    pltpu.async_copy(hs_hbm.at[idx_v.at[g + 1]], nrows,
                                     nsem)

                pltpu.make_async_copy(hs_hbm.at[idx_v.at[g]], rows,
                                      sem).wait()
                lax.fori_loop(0, GN, make_node_body(rows), 0)
                pltpu.sync_copy(out_v, out_hbm.at[pl.ds(base + g * GN, GN)])
            return carry

        lax.fori_loop(0, ng // 2, pair_body, 0)

    return agg


def kernel(x, positions, topk, W, a_src, a_dst, ln_gamma, ln_beta):
    B, N, F = x.shape
    H, D = a_src.shape
    npad = ((N + 255) // 256) * 256

    x2 = jnp.pad(x[0], ((0, npad - N), (0, 0)))
    pos = jnp.pad(positions[0], ((0, npad - N), (0, 0)))
    sq = jnp.sum(pos * pos, axis=1)
    sqcol = jnp.where(jnp.arange(npad) < N, sq, 1e30)
    colinfo = jnp.concatenate(
        [pos.T, sqcol[None, :], jnp.zeros((4, npad), jnp.float32)], axis=0)
    rowinfo = jnp.concatenate(
        [pos, sq[:, None], jnp.zeros((npad, 4), jnp.float32)], axis=1)

    avec = (a_src + a_dst).reshape(H * D)
    sel = (jnp.arange(H * D)[:, None] // D == jnp.arange(H)[None, :])
    W2 = (W * avec[None, :]) @ sel.astype(jnp.float32)
    Wsc = W2[:, jnp.arange(H * D) // D]
    Wbig = jnp.concatenate([W, Wsc], axis=1)

    rb2 = 1024
    hs = pl.pallas_call(
        _proj_body,
        grid=(npad // rb2,),
        in_specs=[
            pl.BlockSpec((rb2, F), lambda i: (i, 0)),
            pl.BlockSpec((F, 256), lambda i: (0, 0)),
        ],
        out_specs=pl.BlockSpec((rb2, 256), lambda i: (i, 0)),
        out_shape=jax.ShapeDtypeStruct((npad, 256), jnp.float32),
    )(x2, Wbig)

    rb = 512
    idx = pl.pallas_call(
        functools.partial(_knn_body, npad=npad, rb=rb),
        grid=(npad // rb,),
        in_specs=[
            pl.BlockSpec((8, npad), lambda i: (0, 0)),
            pl.BlockSpec((rb, 8), lambda i: (i, 0)),
        ],
        out_specs=pl.BlockSpec((rb, K), lambda i: (i, 0)),
        out_shape=jax.ShapeDtypeStruct((npad, K), jnp.int32),
    )(colinfo, rowinfo)

    idxr = idx.reshape(NW, npad // NW // GN, ROWS)
    hp = _make_agg(npad)(hs, idxr)

    g2 = jnp.tile(ln_gamma[None, :], (8, 1))
    b2 = jnp.tile(ln_beta[None, :], (8, 1))
    out = pl.pallas_call(
        _ln_body,
        grid=(npad // rb2,),
        in_specs=[
            pl.BlockSpec((rb2, F), lambda i: (i, 0)),
            pl.BlockSpec((rb2, F), lambda i: (i, 0)),
            pl.BlockSpec((8, F), lambda i: (0, 0)),
            pl.BlockSpec((8, F), lambda i: (0, 0)),
        ],
        out_specs=pl.BlockSpec((rb2, F), lambda i: (i, 0)),
        out_shape=jax.ShapeDtypeStruct((npad, F), jnp.float32),
    )(hp, x2, g2, b2)

    return out[:N].reshape(B, N, H * D)
